# baseline (speedup 1.0000x reference)
import functools

import jax
import jax.numpy as jnp
from jax import lax
from jax.experimental import pallas as pl
from jax.experimental.pallas import tpu as pltpu
from jax.experimental.pallas import tpu_sc as plsc

NC = 2
NS = 16
CH = 64
IB = 16
NB = 4
LN = 16
BLK = 512


def _sc_segment_sums(N_PAD, D, FCH):
    RPT = N_PAD // NS

    mesh = plsc.VectorSubcoreMesh(core_axis_name="c", subcore_axis_name="s")

    @functools.partial(
        pl.kernel,
        out_type=(
            jax.ShapeDtypeStruct((NC, N_PAD, D), jnp.float32),
            jax.ShapeDtypeStruct((NC, NS, N_PAD), jnp.float32),
        ),
        mesh=mesh,
        compiler_params=pltpu.CompilerParams(needs_layout_passes=False),
        scratch_types=[
            pltpu.VMEM_SHARED((N_PAD, D), jnp.float32),
            pltpu.VMEM((IB, CH), jnp.int32),
            pltpu.VMEM((IB, CH), jnp.int32),
            pltpu.VMEM((NB, CH, D), jnp.float32),
            pltpu.VMEM((N_PAD,), jnp.float32),
            [pltpu.SemaphoreType.DMA] * NB,
            [pltpu.SemaphoreType.DMA] * NB,
        ],
    )
    def seg(xh_hbm, src_hbm, dstf_hbm,
            acc_out, hist_out,
            acc_sh, sidx, didx, rows, hist, gsems, ssems):
        c = lax.axis_index("c")
        s = lax.axis_index("s")
        rbase = s * RPT

        zv = jnp.zeros((LN,), jnp.float32)

        def zrow(i, carry):
            def zcol(j, carry2):
                rows[0, i, pl.ds(j * LN, LN)] = zv
                return carry2

            return lax.fori_loop(0, D // LN, zcol, carry)

        lax.fori_loop(0, CH, zrow, 0)

        def zhist(i, carry):
            hist[pl.ds(i * LN, LN)] = zv
            return carry

        lax.fori_loop(0, N_PAD // LN, zhist, 0)

        def zinit(k, carry):
            pltpu.sync_copy(rows.at[0], acc_sh.at[pl.ds(rbase + k * CH, CH)])
            return carry

        lax.fori_loop(0, RPT // CH, zinit, 0)
        plsc.subcore_barrier()

        ones16 = jnp.ones((LN,), jnp.float32)

        def fouter(jo, carry):
            pltpu.sync_copy(src_hbm.at[c, s, pl.ds(jo * IB, IB)], sidx)
            pltpu.sync_copy(dstf_hbm.at[s, pl.ds(jo * IB, IB)], didx)
            pend_g = [
                pltpu.async_copy(xh_hbm.at[sidx.at[q]], rows.at[q], gsems[q])
                for q in range(NB)
            ]
            pend_s = [None] * NB
            for j in range(IB):
                q = j % NB
                pend_g[q].wait()
                pend_s[q] = pltpu.async_copy(
                    rows.at[q], acc_sh.at[didx.at[j]], ssems[q], add=True)
                for k in range(CH // LN):
                    idx = didx[j, pl.ds(k * LN, LN)]
                    plsc.addupdate_scatter(hist, [idx], ones16)
                if j >= NB - 1 and j + 1 < IB:
                    qn = (j + 1) % NB
                    pend_s[qn].wait()
                    pend_g[qn] = pltpu.async_copy(
                        xh_hbm.at[sidx.at[j + 1]], rows.at[qn], gsems[qn])
            for t in range(NB):
                pend_s[(IB - NB + t) % NB].wait()
            return carry

        lax.fori_loop(0, FCH // IB, fouter, 0)

        pltpu.sync_copy(hist, hist_out.at[c, s])
        plsc.subcore_barrier()

        def wback(k, carry):
            r0 = rbase + k * CH
            pltpu.sync_copy(acc_sh.at[pl.ds(r0, CH)], rows.at[0])
            pltpu.sync_copy(rows.at[0], acc_out.at[c, pl.ds(r0, CH)])
            return carry

        lax.fori_loop(0, RPT // CH, wback, 0)

    return seg


def _tc_gates(acc_ref, hist_ref, h_ref, w_ref, b_ref, o_ref):
    D = h_ref.shape[1]
    hs = jnp.concatenate([hist_ref[0], hist_ref[1]], axis=0)
    ones = jnp.full((2 * NS, 1), 0.5, jnp.float32)
    deg = jax.lax.dot_general(hs, ones, (((0,), (0,)), ((), ())),
                              preferred_element_type=jnp.float32)
    r = 1.0 / jnp.maximum(deg, 1.0)
    agg = jnp.concatenate([acc_ref[0] * r, acc_ref[1] * r], axis=1)
    pre = jnp.dot(agg, w_ref[...], preferred_element_type=jnp.float32) + b_ref[...]
    u = jax.nn.sigmoid(pre[:, :D])
    cc = jnp.tanh(pre[:, D:])
    o_ref[...] = u * h_ref[...] + (1.0 - u) * cc


def kernel(x, h, edge_index, W_rx, b_rx, W_rh, b_rh, W_ux, b_ux, W_uh, b_uh,
           W_cx, b_cx, W_ch, b_ch):
    N, D = x.shape
    E = edge_index.shape[1]

    N_PAD = ((N + BLK - 1) // BLK) * BLK
    if N_PAD == N:
        N_PAD += BLK
    EQ = NS * CH * IB
    E_PAD = ((E + EQ - 1) // EQ) * EQ
    FCH = E_PAD // (NS * CH)

    src = edge_index[0]
    dst = edge_index[1]
    pad_e = E_PAD - E
    src_p = jnp.concatenate([src, jnp.zeros((pad_e,), jnp.int32)])
    dst_p = jnp.concatenate([dst, jnp.full((pad_e,), N, jnp.int32)])

    xh = jnp.concatenate([x, h], axis=0)
    src2 = jnp.stack([src_p, src_p + N]).reshape(NC, NS, FCH, CH)
    dstf = dst_p.reshape(NS, FCH, CH)

    acc, hists = _sc_segment_sums(N_PAD, D, FCH)(xh, src2, dstf)

    Wcat = jnp.concatenate(
        [jnp.concatenate([W_ux, W_cx], axis=1),
         jnp.concatenate([W_uh, W_ch], axis=1)], axis=0)
    bcat = jnp.concatenate([b_ux + b_uh, b_cx + b_ch]).reshape(1, 2 * D)
    h_pad = jnp.pad(h, ((0, N_PAD - N), (0, 0)))

    out = pl.pallas_call(
        _tc_gates,
        grid=(N_PAD // BLK,),
        in_specs=[
            pl.BlockSpec((NC, BLK, D), lambda i: (0, i, 0)),
            pl.BlockSpec((NC, NS, BLK), lambda i: (0, 0, i)),
            pl.BlockSpec((BLK, D), lambda i: (i, 0)),
            pl.BlockSpec((2 * D, 2 * D), lambda i: (0, 0)),
            pl.BlockSpec((1, 2 * D), lambda i: (0, 0)),
        ],
        out_specs=pl.BlockSpec((BLK, D), lambda i: (i, 0)),
        out_shape=jax.ShapeDtypeStruct((N_PAD, D), jnp.float32),
    )(acc, hists, h_pad, Wcat, bcat)
    return out[:N]


# speedup vs baseline: 1.3100x; 1.3100x over previous
import functools

import jax
import jax.numpy as jnp
from jax import lax
from jax.experimental import pallas as pl
from jax.experimental.pallas import tpu as pltpu
from jax.experimental.pallas import tpu_sc as plsc

NC = 2
NS = 16
CH = 128
IB = 8
NB = 2
LN = 16
BLK = 512


def _sc_segment_sums(N_PAD, D, FCH):
    RPT = N_PAD // NS

    mesh = plsc.VectorSubcoreMesh(core_axis_name="c", subcore_axis_name="s")

    @functools.partial(
        pl.kernel,
        out_type=(
            jax.ShapeDtypeStruct((NC, N_PAD, D), jnp.float32),
            jax.ShapeDtypeStruct((NC, NS, N_PAD), jnp.float32),
        ),
        mesh=mesh,
        compiler_params=pltpu.CompilerParams(needs_layout_passes=False),
        scratch_types=[
            pltpu.VMEM_SHARED((N_PAD, D), jnp.float32),
            pltpu.VMEM((2 * IB, CH // 2), jnp.int32),
            pltpu.VMEM((IB, CH), jnp.int32),
            pltpu.VMEM((NB, CH, D), jnp.float32),
            pltpu.VMEM((N_PAD,), jnp.float32),
            [pltpu.SemaphoreType.DMA] * NB,
            [pltpu.SemaphoreType.DMA] * NB,
        ],
    )
    def seg(xh_hbm, src_hbm, dstf_hbm,
            acc_out, hist_out,
            acc_sh, sidx, didx, rows, hist, gsems, ssems):
        c = lax.axis_index("c")
        s = lax.axis_index("s")
        rbase = s * RPT

        zv = jnp.zeros((LN,), jnp.float32)

        def zrow(i, carry):
            def zcol(j, carry2):
                rows[0, i, pl.ds(j * LN, LN)] = zv
                return carry2

            return lax.fori_loop(0, D // LN, zcol, carry)

        lax.fori_loop(0, CH, zrow, 0)

        def zhist(i, carry):
            hist[pl.ds(i * LN, LN)] = zv
            return carry

        lax.fori_loop(0, N_PAD // LN, zhist, 0)

        def zinit(k, carry):
            pltpu.sync_copy(rows.at[0], acc_sh.at[pl.ds(rbase + k * CH, CH)])
            return carry

        lax.fori_loop(0, RPT // CH, zinit, 0)
        plsc.subcore_barrier()

        ones16 = jnp.ones((LN,), jnp.float32)

        def fouter(jo, carry):
            pltpu.sync_copy(src_hbm.at[c, s, pl.ds(jo * 2 * IB, 2 * IB)],
                            sidx)
            pltpu.sync_copy(dstf_hbm.at[s, pl.ds(jo * IB, IB)], didx)

            def gat(t, q):
                return (
                    pltpu.async_copy(xh_hbm.at[sidx.at[2 * t]],
                                     rows.at[q, pl.ds(0, CH // 2)],
                                     gsems[q]),
                    pltpu.async_copy(xh_hbm.at[sidx.at[2 * t + 1]],
                                     rows.at[q, pl.ds(CH // 2, CH // 2)],
                                     gsems[q]),
                )

            pend_g = [gat(q, q) for q in range(NB)]
            pend_s = [None] * NB
            for j in range(IB):
                q = j % NB
                pend_g[q][0].wait()
                pend_g[q][1].wait()
                pend_s[q] = pltpu.async_copy(
                    rows.at[q], acc_sh.at[didx.at[j]], ssems[q], add=True)
                for k in range(CH // LN):
                    idx = didx[j, pl.ds(k * LN, LN)]
                    plsc.addupdate_scatter(hist, [idx], ones16)
                if j >= NB - 1 and j + 1 < IB:
                    qn = (j + 1) % NB
                    pend_s[qn].wait()
                    pend_g[qn] = gat(j + 1, qn)
            for t in range(NB):
                pend_s[(IB - NB + t) % NB].wait()
            return carry

        lax.fori_loop(0, FCH // IB, fouter, 0)

        pltpu.sync_copy(hist, hist_out.at[c, s])
        plsc.subcore_barrier()

        def wback(k, carry):
            r0 = rbase + k * CH
            pltpu.sync_copy(acc_sh.at[pl.ds(r0, CH)], rows.at[0])
            pltpu.sync_copy(rows.at[0], acc_out.at[c, pl.ds(r0, CH)])
            return carry

        lax.fori_loop(0, RPT // CH, wback, 0)

    return seg


def _tc_gates(acc_ref, hist_ref, h_ref, w_ref, b_ref, o_ref):
    D = h_ref.shape[1]
    hs = jnp.concatenate([hist_ref[0], hist_ref[1]], axis=0)
    ones = jnp.full((2 * NS, 1), 0.5, jnp.float32)
    deg = jax.lax.dot_general(hs, ones, (((0,), (0,)), ((), ())),
                              preferred_element_type=jnp.float32)
    r = 1.0 / jnp.maximum(deg, 1.0)
    agg = jnp.concatenate([acc_ref[0] * r, acc_ref[1] * r], axis=1)
    pre = jnp.dot(agg, w_ref[...], preferred_element_type=jnp.float32) + b_ref[...]
    u = jax.nn.sigmoid(pre[:, :D])
    cc = jnp.tanh(pre[:, D:])
    o_ref[...] = u * h_ref[...] + (1.0 - u) * cc


def kernel(x, h, edge_index, W_rx, b_rx, W_rh, b_rh, W_ux, b_ux, W_uh, b_uh,
           W_cx, b_cx, W_ch, b_ch):
    N, D = x.shape
    E = edge_index.shape[1]

    N_PAD = ((N + BLK - 1) // BLK) * BLK
    if N_PAD == N:
        N_PAD += BLK
    EQ = NS * CH * IB
    E_PAD = ((E + EQ - 1) // EQ) * EQ
    FCH = E_PAD // (NS * CH)

    src = edge_index[0]
    dst = edge_index[1]
    pad_e = E_PAD - E
    src_p = jnp.concatenate([src, jnp.zeros((pad_e,), jnp.int32)])
    dst_p = jnp.concatenate([dst, jnp.full((pad_e,), N, jnp.int32)])

    xh = jnp.concatenate([x, h], axis=0)
    src2 = jnp.stack([src_p, src_p + N]).reshape(NC, NS, 2 * FCH, CH // 2)
    dstf = dst_p.reshape(NS, FCH, CH)

    acc, hists = _sc_segment_sums(N_PAD, D, FCH)(xh, src2, dstf)

    Wcat = jnp.concatenate(
        [jnp.concatenate([W_ux, W_cx], axis=1),
         jnp.concatenate([W_uh, W_ch], axis=1)], axis=0)
    bcat = jnp.concatenate([b_ux + b_uh, b_cx + b_ch]).reshape(1, 2 * D)
    h_pad = jnp.pad(h, ((0, N_PAD - N), (0, 0)))

    out = pl.pallas_call(
        _tc_gates,
        grid=(N_PAD // BLK,),
        in_specs=[
            pl.BlockSpec((NC, BLK, D), lambda i: (0, i, 0)),
            pl.BlockSpec((NC, NS, BLK), lambda i: (0, 0, i)),
            pl.BlockSpec((BLK, D), lambda i: (i, 0)),
            pl.BlockSpec((2 * D, 2 * D), lambda i: (0, 0)),
            pl.BlockSpec((1, 2 * D), lambda i: (0, 0)),
        ],
        out_specs=pl.BlockSpec((BLK, D), lambda i: (i, 0)),
        out_shape=jax.ShapeDtypeStruct((N_PAD, D), jnp.float32),
    )(acc, hists, h_pad, Wcat, bcat)
    return out[:N]
